# baseline (speedup 1.0000x reference)
import functools

import jax
import jax.numpy as jnp
from jax import lax
from jax.experimental import pallas as pl
from jax.experimental.pallas import tpu as pltpu
from jax.experimental.pallas import tpu_sc as plsc

NC, NS = 2, 16
NW = NC * NS
B, H, D = 4096, 50, 64
BPW = B // NW
G = 2
CH = BPW // G
GH = 104
LANES = 16
LG = D // LANES

_mesh = plsc.VectorSubcoreMesh(core_axis_name="c", subcore_axis_name="s",
                               num_cores=NC, num_subcores=NS)


@functools.partial(
    pl.kernel,
    out_type=jax.ShapeDtypeStruct((B, D), jnp.float32),
    mesh=_mesh,
    compiler_params=pltpu.CompilerParams(use_tc_tiling_on_sc=False),
    scratch_types=[
        pltpu.VMEM((CH, GH), jnp.int32),
        pltpu.VMEM((GH, D), jnp.float32),
        pltpu.VMEM((GH, D), jnp.float32),
        pltpu.VMEM((BPW, D), jnp.float32),
        pltpu.SemaphoreType.DMA,
        pltpu.SemaphoreType.DMA,
    ],
)
def _sc_embed_sum(q_hbm, table_hbm, out_hbm, idx_v, buf0, buf1, out_v,
                  sem0, sem1):
    wid = lax.axis_index("s") * NC + lax.axis_index("c")
    pltpu.sync_copy(q_hbm.at[wid], idx_v)

    def start(g, buf, sem):
        pltpu.async_copy(table_hbm.at[idx_v.at[g]], buf, sem)

    def wait(buf, sem):
        pltpu.make_async_copy(table_hbm.at[pl.ds(0, GH)], buf, sem).wait()

    def accum(buf, g):
        for e in range(G):
            for l in range(LG):
                acc = buf[e * H, pl.ds(l * LANES, LANES)]
                for r in range(1, H):
                    acc = acc + buf[e * H + r, pl.ds(l * LANES, LANES)]
                out_v[g * G + e, pl.ds(l * LANES, LANES)] = acc

    start(0, buf0, sem0)

    def body(i, carry):
        g = 2 * i
        start(g + 1, buf1, sem1)
        wait(buf0, sem0)
        accum(buf0, g)

        @pl.when(g + 2 < CH)
        def _():
            start(g + 2, buf0, sem0)

        wait(buf1, sem1)
        accum(buf1, g + 1)
        return carry

    lax.fori_loop(0, CH // 2, body, 0)
    pltpu.sync_copy(out_v, out_hbm.at[pl.ds(wid * BPW, BPW)])


def kernel(query, table):
    q = query.reshape(NW, CH, G * H)
    q = jnp.pad(q, ((0, 0), (0, 0), (0, GH - G * H)))
    out = _sc_embed_sum(q, table)
    return out[:, None, :]
